# baseline (speedup 1.0000x reference)
import functools

import jax
import jax.numpy as jnp
from jax import lax
from jax.experimental import pallas as pl
from jax.experimental.pallas import tpu as pltpu
from jax.experimental.pallas import tpu_sc as plsc

N = 10000
NPAD = 10240
E = 160000
EPAD = 163840
IN_DIM = 64
H_DIM = 256
FC = 128
NCHUNK = 4
RESIDUAL = 0.12

NC, NS = 2, 16
EPT = EPAD // (NC * NS)
NB = EPT // 128
STRIPE = NPAD // NS

_mesh = plsc.VectorSubcoreMesh(core_axis_name="c", subcore_axis_name="s")


@functools.partial(
    pl.kernel,
    out_type=jax.ShapeDtypeStruct((NC * NPAD, 128), jnp.float32),
    mesh=_mesh,
    scratch_types=[
        pltpu.VMEM((NB, 128), jnp.int32),
        pltpu.VMEM((128, 128), jnp.float32),
        pltpu.VMEM_SHARED((NPAD, 128), jnp.float32),
        pltpu.SemaphoreType.DMA,
    ],
)
def _deg_kernel(dst32_hbm, ones_hbm, zeros_hbm, deg_hbm, idx_v, ones_v, acc_sh,
                sem):
    c = lax.axis_index("c")
    s = lax.axis_index("s")
    r0 = s * STRIPE
    pltpu.sync_copy(zeros_hbm.at[pl.ds(r0, STRIPE)],
                    acc_sh.at[pl.ds(r0, STRIPE)])
    pltpu.sync_copy(dst32_hbm.at[c * NS + s], idx_v)
    pltpu.sync_copy(ones_hbm, ones_v)
    plsc.subcore_barrier()

    def body(b, _):
        pltpu.async_copy(ones_v, acc_sh.at[idx_v.at[b]], sem, add=True)
        return 0

    lax.fori_loop(0, NB, body, 0)

    def drain(b, _):
        pltpu.make_async_copy(ones_v, acc_sh.at[idx_v.at[b]], sem).wait()
        return 0

    lax.fori_loop(0, NB, drain, 0)
    plsc.subcore_barrier()
    pltpu.sync_copy(acc_sh.at[pl.ds(r0, STRIPE)],
                    deg_hbm.at[pl.ds(c * NPAD + r0, STRIPE)])


NBUF = 2


@functools.partial(
    pl.kernel,
    out_type=jax.ShapeDtypeStruct((NCHUNK * NC * NPAD, FC), jnp.float32),
    mesh=_mesh,
    scratch_types=[
        pltpu.VMEM((NB, 128), jnp.int32),
        pltpu.VMEM((NB, 128), jnp.int32),
        pltpu.VMEM((128, FC), jnp.float32),
        pltpu.VMEM((128, FC), jnp.float32),
        pltpu.VMEM_SHARED((NPAD, FC), jnp.float32),
        pltpu.SemaphoreType.DMA,
        pltpu.SemaphoreType.DMA,
        pltpu.SemaphoreType.DMA,
        pltpu.SemaphoreType.DMA,
    ],
)
def _agg_kernel(table_hbm, src4_hbm, dst_hbm, zeros_hbm, out_hbm,
                idxs_v, idxd_v, rb0, rb1, acc_sh, gs0, gs1, ss0, ss1):
    bufs = (rb0, rb1)
    gsems = (gs0, gs1)
    ssems = (ss0, ss1)
    c = lax.axis_index("c")
    s = lax.axis_index("s")
    w = c * NS + s
    r0 = s * STRIPE
    pltpu.sync_copy(dst_hbm.at[w], idxd_v)

    for k in range(NCHUNK):
        pltpu.sync_copy(zeros_hbm.at[pl.ds(r0, STRIPE)],
                        acc_sh.at[pl.ds(r0, STRIPE)])
        pltpu.sync_copy(src4_hbm.at[k, w], idxs_v)
        plsc.subcore_barrier()

        pltpu.async_copy(table_hbm.at[idxs_v.at[0]], bufs[0], gsems[0])

        def grp(g, _):
            for b in range(NBUF):
                i = g * NBUF + b
                pv = (b + NBUF - 1) % NBUF
                pltpu.make_async_copy(table_hbm.at[idxs_v.at[i]],
                                      bufs[b], gsems[b]).wait()

                def _wait_prev_scatter():
                    pltpu.make_async_copy(bufs[pv],
                                          acc_sh.at[idxd_v.at[i - 1]],
                                          ssems[pv]).wait()

                if b == 0:
                    pl.when(i >= 1)(_wait_prev_scatter)
                else:
                    _wait_prev_scatter()

                @pl.when(i + 1 < NB)
                def _():
                    pltpu.async_copy(table_hbm.at[idxs_v.at[i + 1]],
                                     bufs[pv], gsems[pv])

                pltpu.async_copy(bufs[b], acc_sh.at[idxd_v.at[i]],
                                 ssems[b], add=True)
            return 0

        lax.fori_loop(0, NB // NBUF, grp, 0)
        pltpu.make_async_copy(bufs[(NB - 1) % NBUF],
                              acc_sh.at[idxd_v.at[NB - 1]],
                              ssems[(NB - 1) % NBUF]).wait()
        plsc.subcore_barrier()
        pltpu.sync_copy(acc_sh.at[pl.ds(r0, STRIPE)],
                        out_hbm.at[pl.ds((k * NC + c) * NPAD + r0, STRIPE)])
        plsc.subcore_barrier()


def _norm_body(deg_ref, out_ref):
    d = deg_ref[0, :, 0:16] + deg_ref[1, :, 0:16]
    out_ref[...] = jnp.where(d > 0.0, lax.rsqrt(jnp.maximum(d, 1.0)), 0.0)


def _norm_from_deg(norm_ref):
    return norm_ref[:, 0:1]


def _proj_body(text_ref, img_ref, l1_ref, l2_ref, pt_ref, pv_ref):
    pt_ref[...] = jnp.dot(text_ref[...], l1_ref[...],
                          preferred_element_type=jnp.float32)
    pv_ref[...] = jnp.dot(img_ref[...], l2_ref[...],
                          preferred_element_type=jnp.float32)


def _hw0_body(x2_ref, w0_ref, deg_ref, out_ref):
    norm = _norm_from_deg(deg_ref)
    out_ref[0] = jnp.dot(x2_ref[0], w0_ref[...],
                         preferred_element_type=jnp.float32) * norm


def _sum2_body(p_ref, out_ref):
    out_ref[0] = p_ref[0, 0] + p_ref[0, 1]


def _hw1_body(agg_ref, w1_ref, deg_ref, out_ref):
    norm = _norm_from_deg(deg_ref)
    y0a = jax.nn.relu(agg_ref[0] * norm)
    y0b = jax.nn.relu(agg_ref[1] * norm)
    acc = jnp.dot(y0a, w1_ref[0:FC, :], preferred_element_type=jnp.float32)
    acc += jnp.dot(y0b, w1_ref[FC:2 * FC, :], preferred_element_type=jnp.float32)
    out_ref[0] = acc * norm


def _final_body(agg0_ref, agg1_ref, deg_ref, gamma_ref, beta_ref, out_ref):
    norm = _norm_from_deg(deg_ref)
    y0 = jax.nn.relu(agg0_ref[0] * norm)
    y1 = jax.nn.relu(agg1_ref[0] * norm) + RESIDUAL * y0
    h = (y0 + y1)[0:N, :]
    s1 = jnp.sum(h, axis=0)
    s2 = jnp.sum(h * h, axis=0)
    mean = s1 * (1.0 / N)
    var = s2 * (1.0 / N) - mean * mean
    inv = lax.rsqrt(var + 1e-5)
    out_ref[...] = (h - mean) * inv * gamma_ref[0, 0] + beta_ref[0, 0]


def kernel(edge_index, preference_t, preference_v, text_item, img_item,
           linear1, linear2, W0, W1, gamma, beta):
    f32 = jnp.float32
    src = edge_index[0]
    dst = edge_index[1]
    pad = N + (jnp.arange(EPAD - E, dtype=jnp.int32) % (NPAD - N))
    src_pad = jnp.concatenate([src, pad])
    dst_pad = jnp.concatenate([dst, pad])
    dst32 = dst_pad.reshape(NC * NS, NB, 128)
    chunk_off = (jnp.arange(NCHUNK, dtype=jnp.int32) * NPAD)[:, None]
    src4 = (src_pad[None, :] + chunk_off).reshape(NCHUNK, NC * NS, NB, 128)

    zeros_pad = jnp.zeros((NPAD, FC), f32)
    ones128 = jnp.ones((128, 128), f32)

    deg2 = _deg_kernel(dst32, ones128, zeros_pad).reshape(NC, NPAD, 128)
    deg16 = pl.pallas_call(
        _norm_body,
        out_shape=jax.ShapeDtypeStruct((NPAD, 16), f32),
    )(deg2)

    pt, pv = pl.pallas_call(
        _proj_body,
        out_shape=(jax.ShapeDtypeStruct((3962, IN_DIM), f32),
                   jax.ShapeDtypeStruct((3962, IN_DIM), f32)),
    )(text_item, img_item, linear1, linear2)

    zrow = jnp.zeros((NPAD - N, IN_DIM), f32)
    x_text = jnp.concatenate([preference_t, pt, zrow])
    x_img = jnp.concatenate([preference_v, pv, zrow])
    x2 = jnp.stack([x_text, x_img])

    hw0n = pl.pallas_call(
        _hw0_body,
        grid=(NCHUNK,),
        in_specs=[
            pl.BlockSpec((1, NPAD, IN_DIM), lambda c: (c // 2, 0, 0)),
            pl.BlockSpec((IN_DIM, FC), lambda c: (0, c % 2)),
            pl.BlockSpec((NPAD, 16), lambda c: (0, 0)),
        ],
        out_specs=pl.BlockSpec((1, NPAD, FC), lambda c: (c, 0, 0)),
        out_shape=jax.ShapeDtypeStruct((NCHUNK, NPAD, FC), f32),
    )(x2, W0, deg16)

    sum2 = pl.pallas_call(
        _sum2_body,
        grid=(NCHUNK,),
        in_specs=[pl.BlockSpec((1, NC, NPAD, FC), lambda c: (c, 0, 0, 0))],
        out_specs=pl.BlockSpec((1, NPAD, FC), lambda c: (c, 0, 0)),
        out_shape=jax.ShapeDtypeStruct((NCHUNK, NPAD, FC), f32),
    )

    agg0 = _agg_kernel(hw0n.reshape(NCHUNK * NPAD, FC), src4, dst32, zeros_pad)
    agg0 = sum2(agg0.reshape(NCHUNK, NC, NPAD, FC))

    hw1n = pl.pallas_call(
        _hw1_body,
        grid=(NCHUNK,),
        in_specs=[
            pl.BlockSpec((2, NPAD, FC), lambda c: (c // 2, 0, 0)),
            pl.BlockSpec((2 * FC, FC), lambda c: (0, c % 2)),
            pl.BlockSpec((NPAD, 16), lambda c: (0, 0)),
        ],
        out_specs=pl.BlockSpec((1, NPAD, FC), lambda c: (c, 0, 0)),
        out_shape=jax.ShapeDtypeStruct((NCHUNK, NPAD, FC), f32),
    )(agg0, W1, deg16)

    agg1 = _agg_kernel(hw1n.reshape(NCHUNK * NPAD, FC), src4, dst32, zeros_pad)
    agg1 = sum2(agg1.reshape(NCHUNK, NC, NPAD, FC))

    h = pl.pallas_call(
        _final_body,
        grid=(NCHUNK,),
        in_specs=[
            pl.BlockSpec((1, NPAD, FC), lambda c: (c, 0, 0)),
            pl.BlockSpec((1, NPAD, FC), lambda c: (c, 0, 0)),
            pl.BlockSpec((NPAD, 16), lambda c: (0, 0)),
            pl.BlockSpec((1, 1, FC), lambda c: (c, 0, 0)),
            pl.BlockSpec((1, 1, FC), lambda c: (c, 0, 0)),
        ],
        out_specs=pl.BlockSpec((N, FC), lambda c: (0, c)),
        out_shape=jax.ShapeDtypeStruct((N, 2 * H_DIM), f32),
    )(agg0, agg1, deg16, gamma.reshape(NCHUNK, 1, FC), beta.reshape(NCHUNK, 1, FC))

    return h


# speedup vs baseline: 1.4557x; 1.4557x over previous
import functools

import jax
import jax.numpy as jnp
from jax import lax
from jax.experimental import pallas as pl
from jax.experimental.pallas import tpu as pltpu
from jax.experimental.pallas import tpu_sc as plsc

N = 10000
NPAD = 10240
E = 160000
EPAD = 163840
IN_DIM = 64
H_DIM = 256
FC = 128
NCHUNK = 4
RESIDUAL = 0.12

NC, NS = 2, 16
EPT = EPAD // (NC * NS)
NB = EPT // 128
STRIPE = NPAD // NS

_mesh = plsc.VectorSubcoreMesh(core_axis_name="c", subcore_axis_name="s")


@functools.partial(
    pl.kernel,
    out_type=jax.ShapeDtypeStruct((NC * NPAD, 128), jnp.float32),
    mesh=_mesh,
    scratch_types=[
        pltpu.VMEM((NB, 128), jnp.int32),
        pltpu.VMEM((128, 128), jnp.float32),
        pltpu.VMEM_SHARED((NPAD, 128), jnp.float32),
        pltpu.SemaphoreType.DMA,
    ],
)
def _deg_kernel(dst32_hbm, ones_hbm, zeros_hbm, deg_hbm, idx_v, ones_v, acc_sh,
                sem):
    c = lax.axis_index("c")
    s = lax.axis_index("s")
    r0 = s * STRIPE
    pltpu.sync_copy(zeros_hbm.at[pl.ds(r0, STRIPE)],
                    acc_sh.at[pl.ds(r0, STRIPE)])
    pltpu.sync_copy(dst32_hbm.at[c * NS + s], idx_v)
    pltpu.sync_copy(ones_hbm, ones_v)
    plsc.subcore_barrier()

    def body(b, _):
        pltpu.async_copy(ones_v, acc_sh.at[idx_v.at[b]], sem, add=True)
        return 0

    lax.fori_loop(0, NB, body, 0)

    def drain(b, _):
        pltpu.make_async_copy(ones_v, acc_sh.at[idx_v.at[b]], sem).wait()
        return 0

    lax.fori_loop(0, NB, drain, 0)
    plsc.subcore_barrier()
    pltpu.sync_copy(acc_sh.at[pl.ds(r0, STRIPE)],
                    deg_hbm.at[pl.ds(c * NPAD + r0, STRIPE)])


NBUF = 2


def _make_agg(nchunk):
  @functools.partial(
      pl.kernel,
      out_type=jax.ShapeDtypeStruct((nchunk * NC * NPAD, FC), jnp.float32),
      mesh=_mesh,
      scratch_types=[
          pltpu.VMEM((NB, 128), jnp.int32),
          pltpu.VMEM((NB, 128), jnp.int32),
          pltpu.VMEM((128, FC), jnp.float32),
          pltpu.VMEM((128, FC), jnp.float32),
          pltpu.VMEM_SHARED((NPAD, FC), jnp.float32),
          pltpu.SemaphoreType.DMA,
          pltpu.SemaphoreType.DMA,
          pltpu.SemaphoreType.DMA,
          pltpu.SemaphoreType.DMA,
      ],
  )
  def _agg_kernel(table_hbm, srcn_hbm, dst_hbm, zeros_hbm, out_hbm,
                  idxs_v, idxd_v, rb0, rb1, acc_sh, gs0, gs1, ss0, ss1):
    bufs = (rb0, rb1)
    gsems = (gs0, gs1)
    ssems = (ss0, ss1)
    c = lax.axis_index("c")
    s = lax.axis_index("s")
    w = c * NS + s
    r0 = s * STRIPE
    pltpu.sync_copy(dst_hbm.at[w], idxd_v)

    for k in range(nchunk):
        pltpu.sync_copy(zeros_hbm.at[pl.ds(r0, STRIPE)],
                        acc_sh.at[pl.ds(r0, STRIPE)])
        pltpu.sync_copy(srcn_hbm.at[k, w], idxs_v)
        plsc.subcore_barrier()

        pltpu.async_copy(table_hbm.at[idxs_v.at[0]], bufs[0], gsems[0])

        def grp(g, _):
            for b in range(NBUF):
                i = g * NBUF + b
                pv = (b + NBUF - 1) % NBUF
                pltpu.make_async_copy(table_hbm.at[idxs_v.at[i]],
                                      bufs[b], gsems[b]).wait()

                def _wait_prev_scatter():
                    pltpu.make_async_copy(bufs[pv],
                                          acc_sh.at[idxd_v.at[i - 1]],
                                          ssems[pv]).wait()

                if b == 0:
                    pl.when(i >= 1)(_wait_prev_scatter)
                else:
                    _wait_prev_scatter()

                @pl.when(i + 1 < NB)
                def _():
                    pltpu.async_copy(table_hbm.at[idxs_v.at[i + 1]],
                                     bufs[pv], gsems[pv])

                pltpu.async_copy(bufs[b], acc_sh.at[idxd_v.at[i]],
                                 ssems[b], add=True)
            return 0

        lax.fori_loop(0, NB // NBUF, grp, 0)
        pltpu.make_async_copy(bufs[(NB - 1) % NBUF],
                              acc_sh.at[idxd_v.at[NB - 1]],
                              ssems[(NB - 1) % NBUF]).wait()
        plsc.subcore_barrier()
        pltpu.sync_copy(acc_sh.at[pl.ds(r0, STRIPE)],
                        out_hbm.at[pl.ds((k * NC + c) * NPAD + r0, STRIPE)])
        plsc.subcore_barrier()

  return _agg_kernel


_agg1_kernel = _make_agg(1)
_agg4_kernel = _make_agg(NCHUNK)


def _norm_body(deg_ref, out_ref):
    d = deg_ref[0, :, 0:16] + deg_ref[1, :, 0:16]
    out_ref[...] = jnp.where(d > 0.0, lax.rsqrt(jnp.maximum(d, 1.0)), 0.0)


def _norm_from_deg(norm_ref):
    return norm_ref[:, 0:1]


def _proj_body(text_ref, img_ref, l1_ref, l2_ref, pt_ref, pv_ref):
    pt_ref[...] = jnp.dot(text_ref[...], l1_ref[...],
                          preferred_element_type=jnp.float32)
    pv_ref[...] = jnp.dot(img_ref[...], l2_ref[...],
                          preferred_element_type=jnp.float32)


def _xn_body(x2_ref, deg_ref, out_ref):
    norm = _norm_from_deg(deg_ref)
    out_ref[...] = jnp.concatenate([x2_ref[0], x2_ref[1]], axis=1) * norm


def _y0_body(aggx_ref, w0_ref, deg_ref, out_ref):
    norm = _norm_from_deg(deg_ref)
    axs = aggx_ref[0] + aggx_ref[1]
    yt = jax.nn.relu(jnp.dot(axs[:, 0:IN_DIM], w0_ref[...],
                             preferred_element_type=jnp.float32) * norm)
    yi = jax.nn.relu(jnp.dot(axs[:, IN_DIM:2 * IN_DIM], w0_ref[...],
                             preferred_element_type=jnp.float32) * norm)
    out_ref[0] = yt[:, 0:FC]
    out_ref[1] = yt[:, FC:2 * FC]
    out_ref[2] = yi[:, 0:FC]
    out_ref[3] = yi[:, FC:2 * FC]


def _hw1_body(y0_ref, w1_ref, deg_ref, out_ref):
    norm = _norm_from_deg(deg_ref)
    acc = jnp.dot(y0_ref[0], w1_ref[0:FC, :], preferred_element_type=jnp.float32)
    acc += jnp.dot(y0_ref[1], w1_ref[FC:2 * FC, :],
                   preferred_element_type=jnp.float32)
    out_ref[0] = acc * norm


def _final_body(y0_ref, agg1_ref, deg_ref, gamma_ref, beta_ref, out_ref):
    norm = _norm_from_deg(deg_ref)
    y0 = y0_ref[0]
    y1 = jax.nn.relu((agg1_ref[0, 0] + agg1_ref[0, 1]) * norm) + RESIDUAL * y0
    h = (y0 + y1)[0:N, :]
    s1 = jnp.sum(h, axis=0)
    s2 = jnp.sum(h * h, axis=0)
    mean = s1 * (1.0 / N)
    var = s2 * (1.0 / N) - mean * mean
    inv = lax.rsqrt(var + 1e-5)
    out_ref[...] = (h - mean) * inv * gamma_ref[0, 0] + beta_ref[0, 0]


def kernel(edge_index, preference_t, preference_v, text_item, img_item,
           linear1, linear2, W0, W1, gamma, beta):
    f32 = jnp.float32
    src = edge_index[0]
    dst = edge_index[1]
    pad = N + (jnp.arange(EPAD - E, dtype=jnp.int32) % (NPAD - N))
    src_pad = jnp.concatenate([src, pad])
    dst_pad = jnp.concatenate([dst, pad])
    dst32 = dst_pad.reshape(NC * NS, NB, 128)
    chunk_off = (jnp.arange(NCHUNK, dtype=jnp.int32) * NPAD)[:, None]
    src4 = (src_pad[None, :] + chunk_off).reshape(NCHUNK, NC * NS, NB, 128)
    src1 = src_pad.reshape(1, NC * NS, NB, 128)

    zeros_pad = jnp.zeros((NPAD, FC), f32)
    ones128 = jnp.ones((128, 128), f32)

    deg2 = _deg_kernel(dst32, ones128, zeros_pad).reshape(NC, NPAD, 128)
    deg16 = pl.pallas_call(
        _norm_body,
        out_shape=jax.ShapeDtypeStruct((NPAD, 16), f32),
    )(deg2)

    pt, pv = pl.pallas_call(
        _proj_body,
        out_shape=(jax.ShapeDtypeStruct((3962, IN_DIM), f32),
                   jax.ShapeDtypeStruct((3962, IN_DIM), f32)),
    )(text_item, img_item, linear1, linear2)

    zrow = jnp.zeros((NPAD - N, IN_DIM), f32)
    x_text = jnp.concatenate([preference_t, pt, zrow])
    x_img = jnp.concatenate([preference_v, pv, zrow])
    x2 = jnp.stack([x_text, x_img])

    xn = pl.pallas_call(
        _xn_body,
        out_shape=jax.ShapeDtypeStruct((NPAD, 2 * IN_DIM), f32),
    )(x2, deg16)

    aggx = _agg1_kernel(xn, src1, dst32, zeros_pad)
    aggx = aggx.reshape(NC, NPAD, FC)

    y0 = pl.pallas_call(
        _y0_body,
        out_shape=jax.ShapeDtypeStruct((NCHUNK, NPAD, FC), f32),
    )(aggx, W0, deg16)

    hw1n = pl.pallas_call(
        _hw1_body,
        grid=(NCHUNK,),
        in_specs=[
            pl.BlockSpec((2, NPAD, FC), lambda c: (c // 2, 0, 0)),
            pl.BlockSpec((2 * FC, FC), lambda c: (0, c % 2)),
            pl.BlockSpec((NPAD, 16), lambda c: (0, 0)),
        ],
        out_specs=pl.BlockSpec((1, NPAD, FC), lambda c: (c, 0, 0)),
        out_shape=jax.ShapeDtypeStruct((NCHUNK, NPAD, FC), f32),
    )(y0, W1, deg16)

    agg1 = _agg4_kernel(hw1n.reshape(NCHUNK * NPAD, FC), src4, dst32, zeros_pad)
    agg1 = agg1.reshape(NCHUNK, NC, NPAD, FC)

    h = pl.pallas_call(
        _final_body,
        grid=(NCHUNK,),
        in_specs=[
            pl.BlockSpec((1, NPAD, FC), lambda c: (c, 0, 0)),
            pl.BlockSpec((1, NC, NPAD, FC), lambda c: (c, 0, 0, 0)),
            pl.BlockSpec((NPAD, 16), lambda c: (0, 0)),
            pl.BlockSpec((1, 1, FC), lambda c: (c, 0, 0)),
            pl.BlockSpec((1, 1, FC), lambda c: (c, 0, 0)),
        ],
        out_specs=pl.BlockSpec((N, FC), lambda c: (0, c)),
        out_shape=jax.ShapeDtypeStruct((N, 2 * H_DIM), f32),
    )(y0, agg1, deg16, gamma.reshape(NCHUNK, 1, FC), beta.reshape(NCHUNK, 1, FC))

    return h


# speedup vs baseline: 1.5104x; 1.0376x over previous
import functools

import jax
import jax.numpy as jnp
from jax import lax
from jax.experimental import pallas as pl
from jax.experimental.pallas import tpu as pltpu
from jax.experimental.pallas import tpu_sc as plsc

N = 10000
NPAD = 10240
E = 160000
EPAD = 163840
IN_DIM = 64
H_DIM = 256
FC = 128
NCHUNK = 4
RESIDUAL = 0.12

NC, NS = 2, 16
EPT = EPAD // (NC * NS)
NB = EPT // 128
STRIPE = NPAD // NS

_mesh = plsc.VectorSubcoreMesh(core_axis_name="c", subcore_axis_name="s")


@functools.partial(
    pl.kernel,
    out_type=jax.ShapeDtypeStruct((NC * NPAD, 128), jnp.float32),
    mesh=_mesh,
    scratch_types=[
        pltpu.VMEM((NB, 128), jnp.int32),
        pltpu.VMEM((128, 128), jnp.float32),
        pltpu.VMEM_SHARED((NPAD, 128), jnp.float32),
        pltpu.SemaphoreType.DMA,
    ],
)
def _deg_kernel(dst32_hbm, ones_hbm, zeros_hbm, deg_hbm, idx_v, ones_v, acc_sh,
                sem):
    c = lax.axis_index("c")
    s = lax.axis_index("s")
    r0 = s * STRIPE
    pltpu.sync_copy(zeros_hbm.at[pl.ds(r0, STRIPE)],
                    acc_sh.at[pl.ds(r0, STRIPE)])
    pltpu.sync_copy(dst32_hbm.at[c * NS + s], idx_v)
    pltpu.sync_copy(ones_hbm, ones_v)
    plsc.subcore_barrier()

    def body(b, _):
        pltpu.async_copy(ones_v, acc_sh.at[idx_v.at[b]], sem, add=True)
        return 0

    lax.fori_loop(0, NB, body, 0)

    def drain(b, _):
        pltpu.make_async_copy(ones_v, acc_sh.at[idx_v.at[b]], sem).wait()
        return 0

    lax.fori_loop(0, NB, drain, 0)
    plsc.subcore_barrier()
    pltpu.sync_copy(acc_sh.at[pl.ds(r0, STRIPE)],
                    deg_hbm.at[pl.ds(c * NPAD + r0, STRIPE)])


NBUF = 2


def _make_agg(nchunk):
  @functools.partial(
      pl.kernel,
      out_type=jax.ShapeDtypeStruct((nchunk * NC * NPAD, FC), jnp.float32),
      mesh=_mesh,
      scratch_types=[
          pltpu.VMEM((NB, 128), jnp.int32),
          pltpu.VMEM((NB, 128), jnp.int32),
          pltpu.VMEM((128, FC), jnp.float32),
          pltpu.VMEM((128, FC), jnp.float32),
          pltpu.VMEM_SHARED((NPAD, FC), jnp.float32),
          pltpu.SemaphoreType.DMA,
          pltpu.SemaphoreType.DMA,
          pltpu.SemaphoreType.DMA,
          pltpu.SemaphoreType.DMA,
      ],
  )
  def _agg_kernel(table_hbm, srcn_hbm, dst_hbm, zeros_hbm, out_hbm,
                  idxs_v, idxd_v, rb0, rb1, acc_sh, gs0, gs1, ss0, ss1):
    bufs = (rb0, rb1)
    gsems = (gs0, gs1)
    ssems = (ss0, ss1)
    c = lax.axis_index("c")
    s = lax.axis_index("s")
    w = c * NS + s
    r0 = s * STRIPE
    pltpu.sync_copy(dst_hbm.at[w], idxd_v)

    for k in range(nchunk):
        pltpu.sync_copy(zeros_hbm.at[pl.ds(r0, STRIPE)],
                        acc_sh.at[pl.ds(r0, STRIPE)])
        pltpu.sync_copy(srcn_hbm.at[k, w], idxs_v)
        plsc.subcore_barrier()

        pltpu.async_copy(table_hbm.at[idxs_v.at[0]], bufs[0], gsems[0])

        def grp(g, _):
            for b in range(NBUF):
                i = g * NBUF + b
                pv = (b + NBUF - 1) % NBUF
                pltpu.make_async_copy(table_hbm.at[idxs_v.at[i]],
                                      bufs[b], gsems[b]).wait()

                def _wait_prev_scatter():
                    pltpu.make_async_copy(bufs[pv],
                                          acc_sh.at[idxd_v.at[i - 1]],
                                          ssems[pv]).wait()

                if b == 0:
                    pl.when(i >= 1)(_wait_prev_scatter)
                else:
                    _wait_prev_scatter()

                @pl.when(i + 1 < NB)
                def _():
                    pltpu.async_copy(table_hbm.at[idxs_v.at[i + 1]],
                                     bufs[pv], gsems[pv])

                pltpu.async_copy(bufs[b], acc_sh.at[idxd_v.at[i]],
                                 ssems[b], add=True)
            return 0

        lax.fori_loop(0, NB // NBUF, grp, 0)
        pltpu.make_async_copy(bufs[(NB - 1) % NBUF],
                              acc_sh.at[idxd_v.at[NB - 1]],
                              ssems[(NB - 1) % NBUF]).wait()
        plsc.subcore_barrier()
        pltpu.sync_copy(acc_sh.at[pl.ds(r0, STRIPE)],
                        out_hbm.at[pl.ds((k * NC + c) * NPAD + r0, STRIPE)])
        plsc.subcore_barrier()

  return _agg_kernel


_agg1_kernel = _make_agg(1)
_agg4_kernel = _make_agg(NCHUNK)


def _norm_body(deg_ref, out_ref):
    d = deg_ref[0, :, 0:16] + deg_ref[1, :, 0:16]
    out_ref[...] = jnp.where(d > 0.0, lax.rsqrt(jnp.maximum(d, 1.0)), 0.0)


def _norm_from_deg(norm_ref):
    return norm_ref[:, 0:1]


def _proj_body(text_ref, img_ref, l1_ref, l2_ref, pt_ref, pv_ref):
    pt_ref[...] = jnp.dot(text_ref[...], l1_ref[...],
                          preferred_element_type=jnp.float32)
    pv_ref[...] = jnp.dot(img_ref[...], l2_ref[...],
                          preferred_element_type=jnp.float32)


def _xn_body(x2_ref, deg2_ref, out_ref):
    d = deg2_ref[0, :, 0:1] + deg2_ref[1, :, 0:1]
    norm = jnp.where(d > 0.0, lax.rsqrt(jnp.maximum(d, 1.0)), 0.0)
    out_ref[...] = jnp.concatenate([x2_ref[0], x2_ref[1]], axis=1) * norm


def _hw1_body(aggx_ref, w0b_ref, w1_ref, deg_ref, out_ref):
    norm = _norm_from_deg(deg_ref)
    axs = aggx_ref[0] + aggx_ref[1]
    y0h = jax.nn.relu(jnp.dot(axs, w0b_ref[0],
                              preferred_element_type=jnp.float32) * norm)
    out_ref[0] = jnp.dot(y0h, w1_ref[...],
                         preferred_element_type=jnp.float32) * norm


def _final_body(aggx_ref, w0sel_ref, agg1_ref, deg_ref, gamma_ref, beta_ref,
                out_ref):
    norm = _norm_from_deg(deg_ref)
    axs = aggx_ref[0] + aggx_ref[1]
    y0 = jax.nn.relu(jnp.dot(axs, w0sel_ref[0],
                             preferred_element_type=jnp.float32) * norm)
    y1 = jax.nn.relu((agg1_ref[0, 0] + agg1_ref[0, 1]) * norm) + RESIDUAL * y0
    h = (y0 + y1)[0:N, :]
    s1 = jnp.sum(h, axis=0)
    s2 = jnp.sum(h * h, axis=0)
    mean = s1 * (1.0 / N)
    var = s2 * (1.0 / N) - mean * mean
    inv = lax.rsqrt(var + 1e-5)
    out_ref[...] = (h - mean) * inv * gamma_ref[0, 0] + beta_ref[0, 0]


def kernel(edge_index, preference_t, preference_v, text_item, img_item,
           linear1, linear2, W0, W1, gamma, beta):
    f32 = jnp.float32
    src = edge_index[0]
    dst = edge_index[1]
    pad = N + (jnp.arange(EPAD - E, dtype=jnp.int32) % (NPAD - N))
    src_pad = jnp.concatenate([src, pad])
    dst_pad = jnp.concatenate([dst, pad])
    dst32 = dst_pad.reshape(NC * NS, NB, 128)
    chunk_off = (jnp.arange(NCHUNK, dtype=jnp.int32) * NPAD)[:, None]
    src4 = (src_pad[None, :] + chunk_off).reshape(NCHUNK, NC * NS, NB, 128)
    src1 = src_pad.reshape(1, NC * NS, NB, 128)

    zeros_pad = jnp.zeros((NPAD, FC), f32)
    ones128 = jnp.ones((128, 128), f32)

    deg2 = _deg_kernel(dst32, ones128, zeros_pad).reshape(NC, NPAD, 128)
    deg16 = pl.pallas_call(
        _norm_body,
        out_shape=jax.ShapeDtypeStruct((NPAD, 16), f32),
    )(deg2)

    pt, pv = pl.pallas_call(
        _proj_body,
        out_shape=(jax.ShapeDtypeStruct((3962, IN_DIM), f32),
                   jax.ShapeDtypeStruct((3962, IN_DIM), f32)),
    )(text_item, img_item, linear1, linear2)

    zrow = jnp.zeros((NPAD - N, IN_DIM), f32)
    x_text = jnp.concatenate([preference_t, pt, zrow])
    x_img = jnp.concatenate([preference_v, pv, zrow])
    x2 = jnp.stack([x_text, x_img])

    xn = pl.pallas_call(
        _xn_body,
        out_shape=jax.ShapeDtypeStruct((NPAD, 2 * IN_DIM), f32),
    )(x2, deg2)

    aggx = _agg1_kernel(xn, src1, dst32, zeros_pad)
    aggx = aggx.reshape(NC, NPAD, FC)

    z64 = jnp.zeros((IN_DIM, 2 * H_DIM // 2), f32)
    w0b = jnp.stack([jnp.concatenate([W0, jnp.zeros_like(W0)], axis=0),
                     jnp.concatenate([jnp.zeros_like(W0), W0], axis=0)])
    w0sel = jnp.stack([w0b[c // 2, :, (c % 2) * FC:(c % 2 + 1) * FC]
                       for c in range(NCHUNK)])

    hw1n = pl.pallas_call(
        _hw1_body,
        grid=(NCHUNK,),
        in_specs=[
            pl.BlockSpec((NC, NPAD, FC), lambda c: (0, 0, 0)),
            pl.BlockSpec((1, 2 * IN_DIM, 2 * FC), lambda c: (c // 2, 0, 0)),
            pl.BlockSpec((2 * FC, FC), lambda c: (0, c % 2)),
            pl.BlockSpec((NPAD, 16), lambda c: (0, 0)),
        ],
        out_specs=pl.BlockSpec((1, NPAD, FC), lambda c: (c, 0, 0)),
        out_shape=jax.ShapeDtypeStruct((NCHUNK, NPAD, FC), f32),
    )(aggx, w0b, W1, deg16)

    agg1 = _agg4_kernel(hw1n.reshape(NCHUNK * NPAD, FC), src4, dst32, zeros_pad)
    agg1 = agg1.reshape(NCHUNK, NC, NPAD, FC)

    h = pl.pallas_call(
        _final_body,
        grid=(NCHUNK,),
        in_specs=[
            pl.BlockSpec((NC, NPAD, FC), lambda c: (0, 0, 0)),
            pl.BlockSpec((1, 2 * IN_DIM, FC), lambda c: (c, 0, 0)),
            pl.BlockSpec((1, NC, NPAD, FC), lambda c: (c, 0, 0, 0)),
            pl.BlockSpec((NPAD, 16), lambda c: (0, 0)),
            pl.BlockSpec((1, 1, FC), lambda c: (c, 0, 0)),
            pl.BlockSpec((1, 1, FC), lambda c: (c, 0, 0)),
        ],
        out_specs=pl.BlockSpec((N, FC), lambda c: (0, c)),
        out_shape=jax.ShapeDtypeStruct((N, 2 * H_DIM), f32),
    )(aggx, w0sel, agg1, deg16,
      gamma.reshape(NCHUNK, 1, FC), beta.reshape(NCHUNK, 1, FC))

    return h
